# baseline (speedup 1.0000x reference)
import jax
import jax.numpy as jnp
from jax.experimental import pallas as pl
from jax.experimental.pallas import tpu as pltpu

_ROWS = 8


def _monotone_key(x):
    xi = jax.lax.bitcast_convert_type(x, jnp.int32)
    return xi ^ (jax.lax.shift_right_arithmetic(xi, 31) & jnp.int32(0x7FFFFFFF))


def _mid(lo, hi):
    return (lo & hi) + jax.lax.shift_right_arithmetic(lo ^ hi, 1)


def _row_body(k_ref, p_ref, x_ref, o_ref):
    kk = k_ref[0]
    pp = p_ref[0]
    x = x_ref[...]
    v = x.shape[1]

    key = _monotone_key(x)
    col = jax.lax.broadcasted_iota(jnp.int32, x.shape, 1)

    def rsum(a):
        return jnp.sum(a, axis=1, keepdims=True)

    def bs_topk(_, c):
        lo, hi = c
        mid = _mid(lo, hi)
        cnt = rsum((key > mid).astype(jnp.int32))
        pred = cnt < kk
        return jnp.where(pred, lo, mid + 1), jnp.where(pred, mid, hi)

    full = jnp.zeros((_ROWS, 1), jnp.int32)
    tk, _ = jax.lax.fori_loop(
        0, 32, bs_topk, (full + jnp.int32(-2147483648), full + jnp.int32(2147483647)))

    m = jnp.max(x, axis=1, keepdims=True)
    e = jnp.where(key >= tk, jnp.exp(x - m), 0.0)
    z = rsum(e)
    pz = pp * z
    kmax = jnp.max(key, axis=1, keepdims=True)

    def bs_topp(_, c):
        lo, hi = c
        mid = _mid(lo, hi)
        g = rsum(jnp.where(key > mid, e, 0.0))
        pred = g < pz
        return jnp.where(pred, lo, mid + 1), jnp.where(pred, mid, hi)

    ub, _ = jax.lax.fori_loop(0, 32, bs_topp, (tk, kmax))

    gv = rsum(jnp.where(key > ub, e, 0.0))
    qe = jnp.max(jnp.where(key == ub, e, 0.0), axis=1, keepdims=True)
    c_eq = rsum((key == ub).astype(jnp.int32))

    def bs_tie(_, c):
        lo, hi = c
        mid = (lo + hi) // 2
        r = rsum(jnp.where((key == ub) & (col > mid), 1.0, 0.0))
        pred = gv + r * qe < pz
        return jnp.where(pred, lo, mid + 1), jnp.where(pred, mid, hi)

    def tie_search():
        i1, _ = jax.lax.fori_loop(0, 17, bs_tie, (full, full + jnp.int32(v - 1)))
        return i1

    i1 = jax.lax.cond(jnp.any(c_eq > 1), tie_search, lambda: full)

    keep = (key > ub) | ((key == ub) & (col >= i1))
    o_ref[...] = jnp.where(keep, x, -jnp.inf)


def kernel(logits, k, p):
    bsz, v = logits.shape
    nb = bsz // _ROWS
    return pl.pallas_call(
        _row_body,
        grid=(nb,),
        in_specs=[
            pl.BlockSpec((1, _ROWS, 1), lambda b: (b, 0, 0)),
            pl.BlockSpec((1, _ROWS, 1), lambda b: (b, 0, 0)),
            pl.BlockSpec((_ROWS, v), lambda b: (b, 0)),
        ],
        out_specs=pl.BlockSpec((_ROWS, v), lambda b: (b, 0)),
        out_shape=jax.ShapeDtypeStruct((bsz, v), logits.dtype),
        compiler_params=pltpu.CompilerParams(
            dimension_semantics=("parallel",)),
    )(k.reshape(nb, _ROWS, 1), p.reshape(nb, _ROWS, 1), logits)


# speedup vs baseline: 1.9396x; 1.9396x over previous
import functools

import jax
import jax.numpy as jnp
from jax.experimental import pallas as pl
from jax.experimental.pallas import tpu as pltpu
from jax.experimental.pallas import tpu_sc as plsc

_ROWS = 8
_CAP = 4096
_CAP_A = 3500
_NC, _NS, _L = 2, 16, 16


def _monotone_key(x):
    xi = jax.lax.bitcast_convert_type(x, jnp.int32)
    return xi ^ (jax.lax.shift_right_arithmetic(xi, 31) & jnp.int32(0x7FFFFFFF))


def _mid(lo, hi):
    return (lo & hi) + jax.lax.shift_right_arithmetic(lo ^ hi, 1)


def _narrow_body(k_ref, x_ref, thr_ref, cnt_ref):
    kk = k_ref[0]
    x = x_ref[...]
    key = _monotone_key(x)

    def rsum(a):
        return jnp.sum(a, axis=1, keepdims=True)

    imin = jnp.full((_ROWS, 1), -2147483648, jnp.int32)
    imax = jnp.full((_ROWS, 1), 2147483647, jnp.int32)
    c0 = jnp.full((_ROWS, 1), x.shape[1], jnp.int32)

    def cond(st):
        lo, hi, thr, c = st
        return jnp.any((c > _CAP_A) & (lo < hi))

    def body(st):
        lo, hi, thr, c = st
        mid = _mid(lo, hi)
        cnt = rsum((key > mid).astype(jnp.int32))
        active = (c > _CAP_A) & (lo < hi)
        pred = cnt < kk
        adv = active & ~pred
        return (jnp.where(adv, mid + 1, lo), jnp.where(active & pred, mid, hi),
                jnp.where(adv, mid, thr), jnp.where(adv, cnt, c))

    _, _, thr, c = jax.lax.while_loop(cond, body, (imin, imax, imin, c0))
    thr_ref[...] = thr
    cnt_ref[...] = c


def _stage_a(logits, k):
    bsz, v = logits.shape
    nb = bsz // _ROWS
    return pl.pallas_call(
        _narrow_body,
        grid=(nb,),
        in_specs=[
            pl.BlockSpec((1, _ROWS, 1), lambda b: (b, 0, 0)),
            pl.BlockSpec((_ROWS, v), lambda b: (b, 0)),
        ],
        out_specs=[
            pl.BlockSpec((_ROWS, 1), lambda b: (b, 0)),
            pl.BlockSpec((_ROWS, 1), lambda b: (b, 0)),
        ],
        out_shape=[
            jax.ShapeDtypeStruct((bsz, 1), jnp.int32),
            jax.ShapeDtypeStruct((bsz, 1), jnp.int32),
        ],
    )(k.reshape(nb, _ROWS, 1), logits)


def _stage_b(logits, thr):
    bsz, v = logits.shape
    rows_per = bsz // (_NC * _NS)
    thr_b = jnp.broadcast_to(thr.reshape(bsz, 1), (bsz, _L)).astype(jnp.int32)
    mesh = plsc.VectorSubcoreMesh(core_axis_name="c", subcore_axis_name="s")

    @functools.partial(
        pl.kernel, mesh=mesh,
        compiler_params=pltpu.CompilerParams(needs_layout_passes=False),
        out_type=[
            jax.ShapeDtypeStruct((bsz, _CAP), jnp.float32),
            jax.ShapeDtypeStruct((bsz, _CAP), jnp.int32),
        ],
        scratch_types=[
            pltpu.VMEM((v,), jnp.float32),
            pltpu.VMEM((_CAP,), jnp.float32),
            pltpu.VMEM((_CAP,), jnp.int32),
            pltpu.VMEM((_L,), jnp.int32),
        ],
    )
    def sc_compact(x_hbm, thr_hbm, vals_hbm, cols_hbm, row_v, vals_v, cols_v, thr_v):
        wid = jax.lax.axis_index("s") * _NC + jax.lax.axis_index("c")
        for rr in range(rows_per):
            row = wid * rows_per + rr
            pltpu.sync_copy(x_hbm.at[row], row_v)
            pltpu.sync_copy(thr_hbm.at[row], thr_v)
            thrv = thr_v[...]

            def step(i, off):
                x = row_v[pl.ds(i * _L, _L)]
                xi = jax.lax.bitcast_convert_type(x, jnp.int32)
                key = xi ^ (jax.lax.shift_right_arithmetic(xi, 31)
                            & jnp.int32(0x7FFFFFFF))
                msk = key > thrv
                inc = plsc.cumsum(msk.astype(jnp.int32))
                idx = off + inc - msk.astype(jnp.int32)
                plsc.store_scatter(vals_v, [idx], x, mask=msk)
                colv = jax.lax.iota(jnp.int32, _L) + i * _L
                plsc.store_scatter(cols_v, [idx], colv, mask=msk)
                c16 = jnp.sum(msk.astype(jnp.int32))
                return jnp.minimum(off + c16, _CAP - _L)

            jax.lax.fori_loop(0, v // _L, step, jnp.int32(0))
            pltpu.sync_copy(vals_v, vals_hbm.at[row])
            pltpu.sync_copy(cols_v, cols_hbm.at[row])

    return sc_compact(logits, thr_b)


def _compact_body(k_ref, p_ref, cnt_ref, vals_ref, cols_ref, ub_ref, i1_ref):
    kk = k_ref[0]
    pp = p_ref[0]
    cnt = cnt_ref[...]
    vals = vals_ref[...]
    cols = cols_ref[...]
    bsz = vals.shape[0]

    pos = jax.lax.broadcasted_iota(jnp.int32, vals.shape, 1)
    valid = pos < cnt
    imin = jnp.full((bsz, 1), -2147483648, jnp.int32)
    imax = jnp.full((bsz, 1), 2147483647, jnp.int32)
    key = jnp.where(valid, _monotone_key(vals), imin)

    def rsum(a):
        return jnp.sum(a, axis=1, keepdims=True)

    def bs_topk(_, c):
        lo, hi = c
        mid = _mid(lo, hi)
        pred = rsum((key > mid).astype(jnp.int32)) < kk
        return jnp.where(pred, lo, mid + 1), jnp.where(pred, mid, hi)

    tk, _ = jax.lax.fori_loop(0, 32, bs_topk, (imin, imax))

    xm = jnp.where(valid, vals, -jnp.inf)
    m = jnp.max(xm, axis=1, keepdims=True)
    e = jnp.where(key >= tk, jnp.exp(xm - m), 0.0)
    z = rsum(e)
    pz = pp * z
    kmax = jnp.max(key, axis=1, keepdims=True)

    def cond2(c):
        lo, hi = c
        return jnp.any(lo < hi)

    def bs_topp(c):
        lo, hi = c
        mid = _mid(lo, hi)
        pred = rsum(jnp.where(key > mid, e, 0.0)) < pz
        return jnp.where(pred, lo, mid + 1), jnp.where(pred, mid, hi)

    ub, _ = jax.lax.while_loop(cond2, bs_topp, (tk, kmax))

    gv = rsum(jnp.where(key > ub, e, 0.0))
    qe = jnp.max(jnp.where(key == ub, e, 0.0), axis=1, keepdims=True)
    c_eq = rsum((key == ub).astype(jnp.int32))

    def bs_tie(_, c):
        lo, hi = c
        mid = (lo + hi) // 2
        r = rsum(jnp.where((key == ub) & (cols > mid), 1.0, 0.0))
        pred = gv + r * qe < pz
        return jnp.where(pred, lo, mid + 1), jnp.where(pred, mid, hi)

    def tie_search():
        i1, _ = jax.lax.fori_loop(
            0, 17, bs_tie,
            (jnp.zeros((bsz, 1), jnp.int32),
             jnp.full((bsz, 1), 131071, jnp.int32)))
        return i1

    i1 = jax.lax.cond(jnp.any(c_eq > 1), tie_search,
                      lambda: jnp.zeros((bsz, 1), jnp.int32))
    ub_ref[...] = ub
    i1_ref[...] = i1


def _stage_c(vals, cols, cnt, k, p):
    bsz = vals.shape[0]
    return pl.pallas_call(
        _compact_body,
        grid=(1,),
        in_specs=[
            pl.BlockSpec((1, bsz, 1), lambda b: (0, 0, 0)),
            pl.BlockSpec((1, bsz, 1), lambda b: (0, 0, 0)),
            pl.BlockSpec((bsz, 1), lambda b: (0, 0)),
            pl.BlockSpec((bsz, _CAP), lambda b: (0, 0)),
            pl.BlockSpec((bsz, _CAP), lambda b: (0, 0)),
        ],
        out_specs=[
            pl.BlockSpec((bsz, 1), lambda b: (0, 0)),
            pl.BlockSpec((bsz, 1), lambda b: (0, 0)),
        ],
        out_shape=[
            jax.ShapeDtypeStruct((bsz, 1), jnp.int32),
            jax.ShapeDtypeStruct((bsz, 1), jnp.int32),
        ],
    )(k.reshape(1, bsz, 1), p.reshape(1, bsz, 1), cnt, vals, cols)


def _mask_body(ub_ref, i1_ref, x_ref, o_ref):
    ub = ub_ref[...]
    i1 = i1_ref[...]
    x = x_ref[...]
    key = _monotone_key(x)
    col = jax.lax.broadcasted_iota(jnp.int32, x.shape, 1)
    keep = (key > ub) | ((key == ub) & (col >= i1))
    o_ref[...] = jnp.where(keep, x, -jnp.inf)


def _stage_d(logits, ub, i1):
    bsz, v = logits.shape
    nb = bsz // _ROWS
    return pl.pallas_call(
        _mask_body,
        grid=(nb,),
        in_specs=[
            pl.BlockSpec((_ROWS, 1), lambda b: (b, 0)),
            pl.BlockSpec((_ROWS, 1), lambda b: (b, 0)),
            pl.BlockSpec((_ROWS, v), lambda b: (b, 0)),
        ],
        out_specs=pl.BlockSpec((_ROWS, v), lambda b: (b, 0)),
        out_shape=jax.ShapeDtypeStruct((bsz, v), logits.dtype),
    )(ub, i1, logits)


def kernel(logits, k, p):
    thr, cnt = _stage_a(logits, k)
    vals, cols = _stage_b(logits, thr)
    ub, i1 = _stage_c(vals, cols, cnt, k, p)
    return _stage_d(logits, ub, i1)


# speedup vs baseline: 2.3241x; 1.1983x over previous
import functools

import jax
import jax.numpy as jnp
from jax.experimental import pallas as pl
from jax.experimental.pallas import tpu as pltpu
from jax.experimental.pallas import tpu_sc as plsc

_ROWS = 8
_SEGCAP = 384
_NC, _NS, _L = 2, 16, 16
_CAP = _L * _SEGCAP
_CAP_A = 3500


def _monotone_key(x):
    xi = jax.lax.bitcast_convert_type(x, jnp.int32)
    return xi ^ (jax.lax.shift_right_arithmetic(xi, 31) & jnp.int32(0x7FFFFFFF))


def _mid(lo, hi):
    return (lo & hi) + jax.lax.shift_right_arithmetic(lo ^ hi, 1)


def _narrow_body(k_ref, x_ref, thr_ref):
    kk = k_ref[0]
    x = x_ref[...]
    key = _monotone_key(x)

    def rsum(a):
        return jnp.sum(a, axis=1, keepdims=True)

    imin = jnp.full((_ROWS, 1), -2147483648, jnp.int32)
    imax = jnp.full((_ROWS, 1), 2147483647, jnp.int32)
    c0 = jnp.full((_ROWS, 1), x.shape[1], jnp.int32)

    def cond(st):
        lo, hi, thr, c = st
        return jnp.any((c > _CAP_A) & (lo < hi))

    def body(st):
        lo, hi, thr, c = st
        mid = _mid(lo, hi)
        cnt = rsum((key > mid).astype(jnp.int32))
        active = (c > _CAP_A) & (lo < hi)
        pred = cnt < kk
        adv = active & ~pred
        return (jnp.where(adv, mid + 1, lo), jnp.where(active & pred, mid, hi),
                jnp.where(adv, mid, thr), jnp.where(adv, cnt, c))

    _, _, thr, c = jax.lax.while_loop(cond, body, (imin, imax, imin, c0))
    thr_ref[...] = thr


def _stage_a(logits, k):
    bsz, v = logits.shape
    nb = bsz // _ROWS
    return pl.pallas_call(
        _narrow_body,
        grid=(nb,),
        in_specs=[
            pl.BlockSpec((1, _ROWS, 1), lambda b: (b, 0, 0)),
            pl.BlockSpec((_ROWS, v), lambda b: (b, 0)),
        ],
        out_specs=pl.BlockSpec((_ROWS, 1), lambda b: (b, 0)),
        out_shape=jax.ShapeDtypeStruct((bsz, 1), jnp.int32),
    )(k.reshape(nb, _ROWS, 1), logits)


def _stage_b(logits, thr):
    bsz, v = logits.shape
    rows_per = bsz // (_NC * _NS)
    thr_b = jnp.broadcast_to(thr.reshape(bsz, 1), (bsz, _L)).astype(jnp.int32)
    mesh = plsc.VectorSubcoreMesh(core_axis_name="c", subcore_axis_name="s")

    @functools.partial(
        pl.kernel, mesh=mesh,
        compiler_params=pltpu.CompilerParams(needs_layout_passes=False),
        out_type=[
            jax.ShapeDtypeStruct((bsz, _CAP), jnp.float32),
            jax.ShapeDtypeStruct((bsz, _CAP), jnp.int32),
        ],
        scratch_types=[
            pltpu.VMEM((v,), jnp.float32),
            pltpu.VMEM((_CAP,), jnp.float32),
            pltpu.VMEM((_CAP,), jnp.int32),
            pltpu.VMEM((_L,), jnp.int32),
        ],
    )
    def sc_compact(x_hbm, thr_hbm, vals_hbm, cols_hbm, row_v, vals_v, cols_v, thr_v):
        wid = jax.lax.axis_index("s") * _NC + jax.lax.axis_index("c")
        lane = jax.lax.iota(jnp.int32, _L)
        neg_inf = jnp.full((_L,), -jnp.inf, jnp.float32)
        for rr in range(rows_per):
            row = wid * rows_per + rr
            pltpu.sync_copy(x_hbm.at[row], row_v)
            pltpu.sync_copy(thr_hbm.at[row], thr_v)
            thrv = thr_v[...]

            def prefill(j, _):
                vals_v[pl.ds(j * _L, _L)] = neg_inf
                return 0

            jax.lax.fori_loop(0, _CAP // _L, prefill, 0)

            seg_end = lane * _SEGCAP + (_SEGCAP - 1)

            def step(i, off_v):
                x = row_v[pl.ds(i * _L, _L)]
                xi = jax.lax.bitcast_convert_type(x, jnp.int32)
                key = xi ^ (jax.lax.shift_right_arithmetic(xi, 31)
                            & jnp.int32(0x7FFFFFFF))
                msk = key > thrv
                idx = jnp.minimum(off_v, seg_end)
                plsc.store_scatter(vals_v, [idx], x, mask=msk)
                colv = lane + i * _L
                plsc.store_scatter(cols_v, [idx], colv, mask=msk)
                return off_v + msk.astype(jnp.int32)

            jax.lax.fori_loop(0, v // _L, step, lane * _SEGCAP)
            pltpu.sync_copy(vals_v, vals_hbm.at[row])
            pltpu.sync_copy(cols_v, cols_hbm.at[row])

    return sc_compact(logits, thr_b)


def _compact_body(k_ref, p_ref, vals_ref, cols_ref, ub_ref, i1_ref):
    kk = k_ref[0]
    pp = p_ref[0]
    vals = vals_ref[...]
    cols = cols_ref[...]
    bsz = vals.shape[0]

    imin = jnp.full((bsz, 1), -2147483648, jnp.int32)
    imax = jnp.full((bsz, 1), 2147483647, jnp.int32)
    key = _monotone_key(vals)

    def rsum(a):
        return jnp.sum(a, axis=1, keepdims=True)

    def bs_topk(_, c):
        lo, hi = c
        mid = _mid(lo, hi)
        pred = rsum((key > mid).astype(jnp.int32)) < kk
        return jnp.where(pred, lo, mid + 1), jnp.where(pred, mid, hi)

    tk, _ = jax.lax.fori_loop(0, 32, bs_topk, (imin, imax))

    m = jnp.max(vals, axis=1, keepdims=True)
    e = jnp.where(key >= tk, jnp.exp(vals - m), 0.0)
    z = rsum(e)
    pz = pp * z
    kmax = jnp.max(key, axis=1, keepdims=True)

    def cond2(c):
        lo, hi = c
        return jnp.any(lo < hi)

    def bs_topp(c):
        lo, hi = c
        mid = _mid(lo, hi)
        pred = rsum(jnp.where(key > mid, e, 0.0)) < pz
        return jnp.where(pred, lo, mid + 1), jnp.where(pred, mid, hi)

    ub, _ = jax.lax.while_loop(cond2, bs_topp, (tk, kmax))

    gv = rsum(jnp.where(key > ub, e, 0.0))
    qe = jnp.max(jnp.where(key == ub, e, 0.0), axis=1, keepdims=True)
    c_eq = rsum((key == ub).astype(jnp.int32))

    def bs_tie(_, c):
        lo, hi = c
        mid = (lo + hi) // 2
        r = rsum(jnp.where((key == ub) & (cols > mid), 1.0, 0.0))
        pred = gv + r * qe < pz
        return jnp.where(pred, lo, mid + 1), jnp.where(pred, mid, hi)

    def tie_search():
        i1, _ = jax.lax.fori_loop(
            0, 17, bs_tie,
            (jnp.zeros((bsz, 1), jnp.int32),
             jnp.full((bsz, 1), 131071, jnp.int32)))
        return i1

    i1 = jax.lax.cond(jnp.any(c_eq > 1), tie_search,
                      lambda: jnp.zeros((bsz, 1), jnp.int32))
    ub_ref[...] = ub
    i1_ref[...] = i1


def _stage_c(vals, cols, k, p):
    bsz = vals.shape[0]
    return pl.pallas_call(
        _compact_body,
        grid=(1,),
        in_specs=[
            pl.BlockSpec((1, bsz, 1), lambda b: (0, 0, 0)),
            pl.BlockSpec((1, bsz, 1), lambda b: (0, 0, 0)),
            pl.BlockSpec((bsz, _CAP), lambda b: (0, 0)),
            pl.BlockSpec((bsz, _CAP), lambda b: (0, 0)),
        ],
        out_specs=[
            pl.BlockSpec((bsz, 1), lambda b: (0, 0)),
            pl.BlockSpec((bsz, 1), lambda b: (0, 0)),
        ],
        out_shape=[
            jax.ShapeDtypeStruct((bsz, 1), jnp.int32),
            jax.ShapeDtypeStruct((bsz, 1), jnp.int32),
        ],
    )(k.reshape(1, bsz, 1), p.reshape(1, bsz, 1), vals, cols)


def _mask_body(ub_ref, i1_ref, x_ref, o_ref):
    ub = ub_ref[...]
    i1 = i1_ref[...]
    x = x_ref[...]
    key = _monotone_key(x)
    col = jax.lax.broadcasted_iota(jnp.int32, x.shape, 1)
    keep = (key > ub) | ((key == ub) & (col >= i1))
    o_ref[...] = jnp.where(keep, x, -jnp.inf)


def _stage_d(logits, ub, i1):
    bsz, v = logits.shape
    nb = bsz // _ROWS
    return pl.pallas_call(
        _mask_body,
        grid=(nb,),
        in_specs=[
            pl.BlockSpec((_ROWS, 1), lambda b: (b, 0)),
            pl.BlockSpec((_ROWS, 1), lambda b: (b, 0)),
            pl.BlockSpec((_ROWS, v), lambda b: (b, 0)),
        ],
        out_specs=pl.BlockSpec((_ROWS, v), lambda b: (b, 0)),
        out_shape=jax.ShapeDtypeStruct((bsz, v), logits.dtype),
    )(ub, i1, logits)


def kernel(logits, k, p):
    thr = _stage_a(logits, k)
    vals, cols = _stage_b(logits, thr)
    ub, i1 = _stage_c(vals, cols, k, p)
    return _stage_d(logits, ub, i1)


# speedup vs baseline: 4.2581x; 1.8321x over previous
import functools

import jax
import jax.numpy as jnp
from jax.experimental import pallas as pl
from jax.experimental.pallas import tpu as pltpu
from jax.experimental.pallas import tpu_sc as plsc

_ROWS = 8
_SEGCAP = 384
_NC, _NS, _L = 2, 16, 16
_CAP = _L * _SEGCAP
_CAP_A = 3500


def _monotone_key(x):
    xi = jax.lax.bitcast_convert_type(x, jnp.int32)
    return xi ^ (jax.lax.shift_right_arithmetic(xi, 31) & jnp.int32(0x7FFFFFFF))


def _mid(lo, hi):
    return (lo & hi) + jax.lax.shift_right_arithmetic(lo ^ hi, 1)


def _narrow_body(k_ref, x_ref, thr_ref):
    kk = k_ref[0]
    x = x_ref[...]
    key = _monotone_key(x)

    def rsum(a):
        return jnp.sum(a, axis=1, keepdims=True)

    imin = jnp.full((_ROWS, 1), -2147483648, jnp.int32)
    imax = jnp.full((_ROWS, 1), 2147483647, jnp.int32)
    c0 = jnp.full((_ROWS, 1), x.shape[1], jnp.int32)

    def cond(st):
        lo, hi, thr, c = st
        return jnp.any((c > _CAP_A) & (lo < hi))

    def body(st):
        lo, hi, thr, c = st
        mid = _mid(lo, hi)
        cnt = rsum((key > mid).astype(jnp.int32))
        active = (c > _CAP_A) & (lo < hi)
        pred = cnt < kk
        adv = active & ~pred
        return (jnp.where(adv, mid + 1, lo), jnp.where(active & pred, mid, hi),
                jnp.where(adv, mid, thr), jnp.where(adv, cnt, c))

    _, _, thr, c = jax.lax.while_loop(cond, body, (imin, imax, imin, c0))
    thr_ref[...] = thr


def _stage_a(logits, k):
    bsz, v = logits.shape
    nb = bsz // _ROWS
    return pl.pallas_call(
        _narrow_body,
        grid=(nb,),
        in_specs=[
            pl.BlockSpec((1, _ROWS, 1), lambda b: (b, 0, 0)),
            pl.BlockSpec((_ROWS, v), lambda b: (b, 0)),
        ],
        out_specs=pl.BlockSpec((_ROWS, 1), lambda b: (b, 0)),
        out_shape=jax.ShapeDtypeStruct((bsz, 1), jnp.int32),
    )(k.reshape(nb, _ROWS, 1), logits)


def _stage_b(logits, thr):
    bsz, v = logits.shape
    rows_per = bsz // (_NC * _NS)
    thr_b = jnp.broadcast_to(thr.reshape(bsz, 1), (bsz, _L)).astype(jnp.int32)
    mesh = plsc.VectorSubcoreMesh(core_axis_name="c", subcore_axis_name="s")

    @functools.partial(
        pl.kernel, mesh=mesh,
        compiler_params=pltpu.CompilerParams(needs_layout_passes=False),
        out_type=[
            jax.ShapeDtypeStruct((bsz, _CAP), jnp.float32),
            jax.ShapeDtypeStruct((bsz, _CAP), jnp.int32),
        ],
        scratch_types=[
            pltpu.VMEM((v,), jnp.float32),
            pltpu.VMEM((_CAP,), jnp.float32),
            pltpu.VMEM((_CAP,), jnp.int32),
            pltpu.VMEM((_L,), jnp.int32),
        ],
    )
    def sc_compact(x_hbm, thr_hbm, vals_hbm, cols_hbm, row_v, vals_v, cols_v, thr_v):
        wid = jax.lax.axis_index("s") * _NC + jax.lax.axis_index("c")
        lane = jax.lax.iota(jnp.int32, _L)
        neg_inf = jnp.full((_L,), -jnp.inf, jnp.float32)
        for rr in range(rows_per):
            row = wid * rows_per + rr
            pltpu.sync_copy(x_hbm.at[row], row_v)
            pltpu.sync_copy(thr_hbm.at[row], thr_v)
            thrv = thr_v[...]

            @functools.partial(plsc.parallel_loop, 0, _CAP // _L, unroll=8)
            def _prefill(j):
                vals_v[pl.ds(j * _L, _L)] = neg_inf

            seg_end = lane * _SEGCAP + (_SEGCAP - 1)

            @functools.partial(plsc.parallel_loop, 0, v // _L, unroll=8,
                               carry=lane * _SEGCAP)
            def _scan(i, off_v):
                x = row_v[pl.ds(i * _L, _L)]
                xi = jax.lax.bitcast_convert_type(x, jnp.int32)
                key = xi ^ (jax.lax.shift_right_arithmetic(xi, 31)
                            & jnp.int32(0x7FFFFFFF))
                msk = key > thrv
                idx = jnp.minimum(off_v, seg_end)
                plsc.store_scatter(vals_v, [idx], x, mask=msk)
                colv = lane + i * _L
                plsc.store_scatter(cols_v, [idx], colv, mask=msk)
                return off_v + msk.astype(jnp.int32)
            pltpu.sync_copy(vals_v, vals_hbm.at[row])
            pltpu.sync_copy(cols_v, cols_hbm.at[row])

    return sc_compact(logits, thr_b)


def _compact_body(k_ref, p_ref, vals_ref, cols_ref, ub_ref, i1_ref):
    kk = k_ref[0]
    pp = p_ref[0]
    vals = vals_ref[...]
    cols = cols_ref[...]
    bsz = vals.shape[0]

    imin = jnp.full((bsz, 1), -2147483648, jnp.int32)
    imax = jnp.full((bsz, 1), 2147483647, jnp.int32)
    key = _monotone_key(vals)

    def rsum(a):
        return jnp.sum(a, axis=1, keepdims=True)

    def bs_topk(_, c):
        lo, hi = c
        mid = _mid(lo, hi)
        pred = rsum((key > mid).astype(jnp.int32)) < kk
        return jnp.where(pred, lo, mid + 1), jnp.where(pred, mid, hi)

    tk, _ = jax.lax.fori_loop(0, 32, bs_topk, (imin, imax))

    m = jnp.max(vals, axis=1, keepdims=True)
    e = jnp.where(key >= tk, jnp.exp(vals - m), 0.0)
    z = rsum(e)
    pz = pp * z
    kmax = jnp.max(key, axis=1, keepdims=True)

    def cond2(c):
        lo, hi = c
        return jnp.any(lo < hi)

    def bs_topp(c):
        lo, hi = c
        mid = _mid(lo, hi)
        pred = rsum(jnp.where(key > mid, e, 0.0)) < pz
        return jnp.where(pred, lo, mid + 1), jnp.where(pred, mid, hi)

    ub, _ = jax.lax.while_loop(cond2, bs_topp, (tk, kmax))

    gv = rsum(jnp.where(key > ub, e, 0.0))
    qe = jnp.max(jnp.where(key == ub, e, 0.0), axis=1, keepdims=True)
    c_eq = rsum((key == ub).astype(jnp.int32))

    def bs_tie(_, c):
        lo, hi = c
        mid = (lo + hi) // 2
        r = rsum(jnp.where((key == ub) & (cols > mid), 1.0, 0.0))
        pred = gv + r * qe < pz
        return jnp.where(pred, lo, mid + 1), jnp.where(pred, mid, hi)

    def tie_search():
        i1, _ = jax.lax.fori_loop(
            0, 17, bs_tie,
            (jnp.zeros((bsz, 1), jnp.int32),
             jnp.full((bsz, 1), 131071, jnp.int32)))
        return i1

    i1 = jax.lax.cond(jnp.any(c_eq > 1), tie_search,
                      lambda: jnp.zeros((bsz, 1), jnp.int32))
    ub_ref[...] = ub
    i1_ref[...] = i1


def _stage_c(vals, cols, k, p):
    bsz = vals.shape[0]
    return pl.pallas_call(
        _compact_body,
        grid=(1,),
        in_specs=[
            pl.BlockSpec((1, bsz, 1), lambda b: (0, 0, 0)),
            pl.BlockSpec((1, bsz, 1), lambda b: (0, 0, 0)),
            pl.BlockSpec((bsz, _CAP), lambda b: (0, 0)),
            pl.BlockSpec((bsz, _CAP), lambda b: (0, 0)),
        ],
        out_specs=[
            pl.BlockSpec((bsz, 1), lambda b: (0, 0)),
            pl.BlockSpec((bsz, 1), lambda b: (0, 0)),
        ],
        out_shape=[
            jax.ShapeDtypeStruct((bsz, 1), jnp.int32),
            jax.ShapeDtypeStruct((bsz, 1), jnp.int32),
        ],
    )(k.reshape(1, bsz, 1), p.reshape(1, bsz, 1), vals, cols)


def _mask_body(ub_ref, i1_ref, x_ref, o_ref):
    ub = ub_ref[...]
    i1 = i1_ref[...]
    x = x_ref[...]
    key = _monotone_key(x)
    col = jax.lax.broadcasted_iota(jnp.int32, x.shape, 1)
    keep = (key > ub) | ((key == ub) & (col >= i1))
    o_ref[...] = jnp.where(keep, x, -jnp.inf)


def _stage_d(logits, ub, i1):
    bsz, v = logits.shape
    nb = bsz // _ROWS
    return pl.pallas_call(
        _mask_body,
        grid=(nb,),
        in_specs=[
            pl.BlockSpec((_ROWS, 1), lambda b: (b, 0)),
            pl.BlockSpec((_ROWS, 1), lambda b: (b, 0)),
            pl.BlockSpec((_ROWS, v), lambda b: (b, 0)),
        ],
        out_specs=pl.BlockSpec((_ROWS, v), lambda b: (b, 0)),
        out_shape=jax.ShapeDtypeStruct((bsz, v), logits.dtype),
    )(ub, i1, logits)


def kernel(logits, k, p):
    thr = _stage_a(logits, k)
    vals, cols = _stage_b(logits, thr)
    ub, i1 = _stage_c(vals, cols, k, p)
    return _stage_d(logits, ub, i1)
